# Initial kernel scaffold; baseline (speedup 1.0000x reference)
import jax
import jax.numpy as jnp
from jax.experimental import pallas as pl


def kernel(i_link, j_link, i_non_link, j_non_link, latent_z, latent_w, bias):
    raise NotImplementedError("write your pallas kernel here")


# speedup vs baseline: 1.8890x; 1.8890x over previous
import functools

import jax
import jax.numpy as jnp
from jax import lax
from jax.experimental import pallas as pl
from jax.experimental.pallas import tpu as pltpu
from jax.experimental.pallas import tpu_sc as plsc

NW = 32
LANES = 16
CHUNK = 128
DIM = 64


def _hinge_partials(i_link, j_link, i_non_link, j_non_link, latent_z,
                    latent_w, bias_vec):
    n_pairs = i_link.shape[0]
    assert n_pairs % CHUNK == 0
    n_chunks = n_pairs // CHUNK
    mesh = plsc.VectorSubcoreMesh(core_axis_name="c", subcore_axis_name="s",
                                  num_cores=2, num_subcores=16)

    @functools.partial(
        pl.kernel,
        mesh=mesh,
        compiler_params=pltpu.CompilerParams(needs_layout_passes=False,
                                             use_tc_tiling_on_sc=False),
        out_type=jax.ShapeDtypeStruct((NW, LANES), jnp.float32),
        scratch_types=[
            pltpu.VMEM((CHUNK,), jnp.int32),
            pltpu.VMEM((CHUNK,), jnp.int32),
            pltpu.VMEM((CHUNK, DIM), jnp.float32),
            pltpu.VMEM((CHUNK, DIM), jnp.float32),
            pltpu.VMEM((LANES,), jnp.float32),
            pltpu.VMEM((LANES,), jnp.float32),
            pltpu.SemaphoreType.DMA,
            pltpu.SemaphoreType.DMA,
        ],
    )
    def body(il_hbm, jl_hbm, inl_hbm, jnl_hbm, z_hbm, w_hbm, bias_hbm,
             out_hbm, idx_i, idx_j, zrows, wrows, biasbuf, accbuf,
             sem_z, sem_w):
        cid = lax.axis_index("c")
        sid = lax.axis_index("s")
        wid = cid * 16 + sid
        pltpu.sync_copy(bias_hbm, biasbuf)
        bv = biasbuf[...]
        iota = lax.iota(jnp.int32, LANES)

        def chunk_hinge(sign):
            def per_chunk(total):
                for q in range(CHUNK // LANES):
                    rows = q * LANES + iota

                    def dstep(dblk, acc):
                        d0 = dblk * 4
                        for dd in range(4):
                            col = jnp.full((LANES,), d0 + dd, jnp.int32)
                            zv = plsc.load_gather(zrows, [rows, col])
                            wv = plsc.load_gather(wrows, [rows, col])
                            df = zv - wv
                            acc = acc + df * df
                        return acc

                    acc = lax.fori_loop(0, DIM // 4, dstep,
                                        jnp.zeros((LANES,), jnp.float32))
                    t = jnp.maximum(acc, 1e-20)
                    ib = lax.bitcast_convert_type(t, jnp.int32)
                    seed = jnp.int32(0x5F3759DF) - lax.shift_right_logical(ib, 1)
                    y = lax.bitcast_convert_type(seed, jnp.float32)
                    for _ in range(3):
                        y = y * (1.5 - 0.5 * t * y * y)
                    dist = t * y
                    h = jnp.maximum(sign * (dist - bv), 0.0)
                    total = total + h
                return total
            return per_chunk

        def make_pass(i_hbm, j_hbm, sign):
            hinge = chunk_hinge(sign)

            def chunk_body(tt, total):
                base = (wid + tt * NW) * CHUNK
                pltpu.sync_copy(i_hbm.at[pl.ds(base, CHUNK)], idx_i)
                pltpu.sync_copy(j_hbm.at[pl.ds(base, CHUNK)], idx_j)
                cz = pltpu.async_copy(z_hbm.at[idx_i], zrows, sem_z)
                cw = pltpu.async_copy(w_hbm.at[idx_j], wrows, sem_w)
                cz.wait()
                cw.wait()
                return hinge(total)
            return chunk_body

        n_w = (n_chunks - wid + NW - 1) // NW
        total = lax.fori_loop(0, n_w, make_pass(il_hbm, jl_hbm, 1.0),
                              jnp.zeros((LANES,), jnp.float32))
        total = lax.fori_loop(0, n_w, make_pass(inl_hbm, jnl_hbm, -1.0),
                              total)
        accbuf[...] = total
        pltpu.sync_copy(accbuf, out_hbm.at[wid])

    return body(i_link, j_link, i_non_link, j_non_link, latent_z, latent_w,
                bias_vec)


def kernel(i_link, j_link, i_non_link, j_non_link, latent_z, latent_w, bias):
    il = i_link.astype(jnp.int32)
    jl = j_link.astype(jnp.int32)
    inl = i_non_link.astype(jnp.int32)
    jnl = j_non_link.astype(jnp.int32)
    bias_vec = jnp.broadcast_to(bias.astype(jnp.float32), (LANES,))
    partials = _hinge_partials(il, jl, inl, jnl, latent_z, latent_w, bias_vec)
    return -jnp.sum(partials)


# speedup vs baseline: 2.3109x; 1.2233x over previous
import functools

import jax
import jax.numpy as jnp
from jax import lax
from jax.experimental import pallas as pl
from jax.experimental.pallas import tpu as pltpu
from jax.experimental.pallas import tpu_sc as plsc

NW = 32
LANES = 16
CHUNK = 128
DIM = 64
SB = 128


def _hinge_partials(idx_i, idx_j, latent_z, latent_w, bias_vec,
                    n_chunks, n_link_chunks):
    base_n = n_chunks // NW
    rem = n_chunks % NW
    mesh = plsc.VectorSubcoreMesh(core_axis_name="c", subcore_axis_name="s",
                                  num_cores=2, num_subcores=16)

    @functools.partial(
        pl.kernel,
        mesh=mesh,
        compiler_params=pltpu.CompilerParams(needs_layout_passes=False,
                                             use_tc_tiling_on_sc=False),
        out_type=jax.ShapeDtypeStruct((NW, LANES), jnp.float32),
        scratch_types=[
            pltpu.VMEM((SB * CHUNK,), jnp.int32),
            pltpu.VMEM((SB * CHUNK,), jnp.int32),
            pltpu.VMEM((CHUNK, DIM), jnp.float32),
            pltpu.VMEM((CHUNK, DIM), jnp.float32),
            pltpu.VMEM((CHUNK, DIM), jnp.float32),
            pltpu.VMEM((CHUNK, DIM), jnp.float32),
            pltpu.VMEM((LANES,), jnp.float32),
            pltpu.VMEM((LANES,), jnp.float32),
            pltpu.SemaphoreType.DMA,
            pltpu.SemaphoreType.DMA,
            pltpu.SemaphoreType.DMA,
            pltpu.SemaphoreType.DMA,
        ],
    )
    def body(ii_hbm, jj_hbm, z_hbm, w_hbm, bias_hbm, out_hbm,
             stg_i, stg_j, zr0, zr1, wr0, wr1, biasbuf, accbuf,
             sz0, sz1, sw0, sw1):
        cid = lax.axis_index("c")
        sid = lax.axis_index("s")
        wid = cid * 16 + sid
        n_w = base_n + jnp.where(wid < rem, 1, 0)
        start_w = wid * base_n + jnp.minimum(wid, rem)
        pltpu.sync_copy(bias_hbm, biasbuf)
        bv = biasbuf[...]
        iota = lax.iota(jnp.int32, LANES)
        accbuf[...] = jnp.zeros((LANES,), jnp.float32)

        zr = (zr0, zr1)
        wr = (wr0, wr1)
        sz = (sz0, sz1)
        sw = (sw0, sw1)

        def gather_descs(tloc, p):
            off = tloc * CHUNK
            cz = pltpu.make_async_copy(
                z_hbm.at[stg_i.at[pl.ds(off, CHUNK)]], zr[p], sz[p])
            cw = pltpu.make_async_copy(
                w_hbm.at[stg_j.at[pl.ds(off, CHUNK)]], wr[p], sw[p])
            return cz, cw

        def compute_chunk(p, sign):
            zrows = zr[p]
            wrows = wr[p]
            for q in range(CHUNK // LANES):
                rows_idx = q * LANES + iota

                def dstep(db, acc):
                    d0 = db * 8
                    for dd in range(8):
                        col = jnp.full((LANES,), d0 + dd, jnp.int32)
                        zv = plsc.load_gather(zrows, [rows_idx, col])
                        wv = plsc.load_gather(wrows, [rows_idx, col])
                        df = zv - wv
                        acc = acc + df * df
                    return acc

                acc = lax.fori_loop(0, DIM // 8, dstep,
                                    jnp.zeros((LANES,), jnp.float32))
                t = jnp.maximum(acc, 1e-20)
                ib = lax.bitcast_convert_type(t, jnp.int32)
                seed = jnp.int32(0x5F3759DF) - lax.shift_right_logical(ib, 1)
                y = lax.bitcast_convert_type(seed, jnp.float32)
                for _ in range(3):
                    y = y * (1.5 - 0.5 * t * y * y)
                dist = t * y
                h = jnp.maximum(sign * (dist - bv), 0.0)
                accbuf[...] = accbuf[...] + h

        def sb_body(sb, _):
            sb_start = start_w + sb * SB
            len_sb = jnp.minimum(SB, n_w - sb * SB)
            pltpu.sync_copy(ii_hbm.at[pl.ds(sb_start * CHUNK, SB * CHUNK)],
                            stg_i)
            pltpu.sync_copy(jj_hbm.at[pl.ds(sb_start * CHUNK, SB * CHUNK)],
                            stg_j)
            cz0, cw0 = gather_descs(0, 0)
            cz0.start()
            cw0.start()

            def u_body(u, __):
                for h_par in (0, 1):
                    t = 2 * u + h_par
                    p = h_par
                    o = 1 - p

                    @pl.when(t < len_sb)
                    def _():
                        @pl.when(t + 1 < len_sb)
                        def _():
                            czn, cwn = gather_descs(t + 1, o)
                            czn.start()
                            cwn.start()
                        czw, cww = gather_descs(t, p)
                        czw.wait()
                        cww.wait()
                        c_glob = sb_start + t
                        sign = jnp.where(c_glob < n_link_chunks, 1.0, -1.0)
                        compute_chunk(p, sign)
                return 0

            lax.fori_loop(0, (len_sb + 1) // 2, u_body, 0)
            return 0

        n_sb = (n_w + SB - 1) // SB
        lax.fori_loop(0, n_sb, sb_body, 0)
        sum_buf = accbuf
        pltpu.sync_copy(sum_buf, out_hbm.at[wid])

    return body(idx_i, idx_j, latent_z, latent_w, bias_vec)


def kernel(i_link, j_link, i_non_link, j_non_link, latent_z, latent_w, bias):
    n_pairs = i_link.shape[0] + i_non_link.shape[0]
    n_chunks = n_pairs // CHUNK
    n_link_chunks = i_link.shape[0] // CHUNK
    assert i_link.shape[0] % CHUNK == 0 and n_pairs % CHUNK == 0
    pad = jnp.zeros((SB * CHUNK,), jnp.int32)
    ii = jnp.concatenate([i_link.astype(jnp.int32),
                          i_non_link.astype(jnp.int32), pad])
    jj = jnp.concatenate([j_link.astype(jnp.int32),
                          j_non_link.astype(jnp.int32), pad])
    bias_vec = jnp.broadcast_to(bias.astype(jnp.float32), (LANES,))
    partials = _hinge_partials(ii, jj, latent_z, latent_w, bias_vec,
                               n_chunks, n_link_chunks)
    return -jnp.sum(partials)
